# baseline (speedup 1.0000x reference)
import jax
import jax.numpy as jnp
from jax.experimental import pallas as pl

B, A, S, C = 64, 3, 52, 80
NC = 5 + C
R = A * S


def _loss_kernel(p0_ref, p1_ref, p2_ref, t0_ref, t1_ref, t2_ref,
                 an_ref, out_ref):
    ident = (jax.lax.broadcasted_iota(jnp.int32, (S, S), 0)
             == jax.lax.broadcasted_iota(jnp.int32, (S, S), 1)
             ).astype(jnp.float32)
    dn = (((1,), (0,)), ((), ()))
    pb = jnp.concatenate(
        [p0_ref[...].reshape(S, S, NC), p1_ref[...].reshape(S, S, NC),
         p2_ref[...].reshape(S, S, NC)], axis=0)[:, :, :8]
    tb = jnp.concatenate(
        [t0_ref[...].reshape(S, S, NC), t1_ref[...].reshape(S, S, NC),
         t2_ref[...].reshape(S, S, NC)], axis=0)[:, :, :8]
    pt = jax.lax.dot_general(pb, ident, dn, preferred_element_type=jnp.float32)
    tt = jax.lax.dot_general(tb, ident, dn, preferred_element_type=jnp.float32)
    p0 = pt[:, 0, :]
    t0 = tt[:, 0, :]
    t1 = tt[:, 1, :]
    t2 = tt[:, 2, :]
    t3 = tt[:, 3, :]
    t4 = tt[:, 4, :]

    row = jax.lax.broadcasted_iota(jnp.int32, (R, S), 0)
    a_idx = row // S
    y = (row % S).astype(jnp.float32)
    x = jax.lax.broadcasted_iota(jnp.int32, (R, S), 1).astype(jnp.float32)

    obj_m = (t0 == 1.0).astype(jnp.float32)
    noobj_m = (t0 == 0.0).astype(jnp.float32)

    noobj_terms = (jnp.maximum(p0, 0.0) - p0 * t0
                   + jnp.log1p(jnp.exp(-jnp.abs(p0))))

    aw = jnp.where(a_idx == 0, an_ref[0, 0],
                   jnp.where(a_idx == 1, an_ref[1, 0], an_ref[2, 0]))
    ah = jnp.where(a_idx == 0, an_ref[0, 1],
                   jnp.where(a_idx == 1, an_ref[1, 1], an_ref[2, 1]))
    bx = jax.nn.sigmoid(t1) + x
    by = jax.nn.sigmoid(t2) + y
    bw = jnp.exp(t3) * aw
    bh = jnp.exp(t4) * ah

    b1x1 = bx - bw * 0.5
    b1y1 = by - bh * 0.5
    b1x2 = bx + bw * 0.5
    b1y2 = by + bh * 0.5
    b2x1 = t1 - t3 * 0.5
    b2y1 = t2 - t4 * 0.5
    b2x2 = t1 + t3 * 0.5
    b2y2 = t2 + t4 * 0.5
    ix1 = jnp.maximum(b1x1, b2x1)
    iy1 = jnp.maximum(b1y1, b2y1)
    ix2 = jnp.minimum(b1x2, b2x2)
    iy2 = jnp.minimum(b1y2, b2y2)
    inter = (jnp.clip(ix2 - ix1, 0.0, None) * jnp.clip(iy2 - iy1, 0.0, None))
    area1 = (b1x2 - b1x1) * (b1y2 - b1y1)
    area2 = (b2x2 - b2x1) * (b2y2 - b2y1)
    union = area1 + area2 - inter + 1e-6
    iou = inter / union
    obj_terms = (jnp.maximum(iou, 0.0) - iou * p0
                 + jnp.log1p(jnp.exp(-jnp.abs(iou))))

    noobj_row = jnp.sum(noobj_terms * noobj_m, axis=0, keepdims=True)
    obj_row = jnp.sum(obj_terms * obj_m, axis=0, keepdims=True)
    k_row = jnp.sum(obj_m, axis=0, keepdims=True)
    n_row = jnp.sum(noobj_m, axis=0, keepdims=True)
    rows = jnp.concatenate(
        [noobj_row, obj_row, k_row, n_row,
         jnp.zeros((4, S), dtype=jnp.float32)], axis=0)
    out_ref[...] = jnp.zeros((8, 128), jnp.float32)
    out_ref[:, 0:S] = rows


@jax.jit
def kernel(predictions, targets, anchors):
    anch = jnp.zeros((8, 128), jnp.float32).at[:A, :2].set(anchors)

    partials = pl.pallas_call(
        _loss_kernel,
        grid=(B,),
        in_specs=(
            [pl.BlockSpec((1, 1, S, S, NC), lambda i, a=a: (i, a, 0, 0, 0))
             for a in range(A)] * 2
            + [pl.BlockSpec((8, 128), lambda i: (0, 0))]
        ),
        out_specs=pl.BlockSpec((None, 8, 128), lambda i: (i, 0, 0)),
        out_shape=jax.ShapeDtypeStruct((B, 8, 128), jnp.float32),
    )(predictions, predictions, predictions, targets, targets, targets,
      anch)

    sums = jnp.sum(partials, axis=(0, 2))
    no_obj_loss = sums[0] / sums[3]
    obj_loss = sums[1] / sums[2]
    return 0.5 * no_obj_loss + obj_loss
